# Initial kernel scaffold; baseline (speedup 1.0000x reference)
import jax
import jax.numpy as jnp
from jax.experimental import pallas as pl


def kernel(anchor, positive, negative):
    raise NotImplementedError("write your pallas kernel here")


# speedup vs baseline: 5286.6544x; 5286.6544x over previous
import jax
import jax.numpy as jnp
from jax.experimental import pallas as pl
from jax.experimental.pallas import tpu as pltpu

_BR = 128
_HALO = 8


def _roll(x, shift, axis):
    return pltpu.roll(x, shift % x.shape[axis], axis)


def _map_body(a_ext, c_ext, o_ref, row0, h, w):
    br = o_ref.shape[0]
    d_ext = jnp.abs(a_ext - c_ext)
    d_m = _roll(d_ext, 1, axis=0)
    c_m = _roll(c_ext, 1, axis=0)

    row_rel = jax.lax.broadcasted_iota(jnp.int32, (br, w), 0)
    col = jax.lax.broadcasted_iota(jnp.int32, (br, w), 1)
    even_r = (row_rel % 2) == 0
    even_c = (col % 2) == 0

    d_k = []
    c_k = []
    for k in range(3):
        d_k.append(jnp.where(even_r, d_ext[k:k + br], d_m[k:k + br]))
        c_k.append(jnp.where(even_r, c_ext[k:k + br], c_m[k:k + br]))

    bd_min = bd_max = bv_min = bv_max = None
    for k in range(3):
        d_sh = {s: _roll(d_k[k], -s, axis=1) for s in (-1, 0, 1, 2)}
        c_sh = {s: _roll(c_k[k], -s, axis=1) for s in (-1, 0, 1, 2)}
        for l in range(3):
            cd = jnp.where(even_c, d_sh[l], d_sh[l - 1])
            cc = jnp.where(even_c, c_sh[l], c_sh[l - 1])
            if bd_min is None:
                bd_min = bd_max = cd
                bv_min = bv_max = cc
            else:
                lt = cd < bd_min
                bd_min = jnp.where(lt, cd, bd_min)
                bv_min = jnp.where(lt, cc, bv_min)
                gt = cd > bd_max
                bd_max = jnp.where(gt, cd, bd_max)
                bv_max = jnp.where(gt, cc, bv_max)

    out = bv_min + bv_max

    row_g = row_rel + row0
    out = jnp.where(col == w - 2, _roll(out, 2, axis=1), out)
    out = jnp.where(row_g == h - 2, _roll(out, 2, axis=0), out)
    c0 = c_ext[0:br]
    out = jnp.where((row_g == h - 1) | (col == w - 1), 2.0 * c0, out)
    o_ref[...] = out


def _dc_kernel(a_ref, p_ref, n_ref, ah_ref, ph_ref, nh_ref, po_ref, no_ref,
               *, h, w, br):
    b = pl.program_id(0)
    row0 = b * br
    a_ext = jnp.concatenate([a_ref[...], ah_ref[...]], axis=0)
    p_ext = jnp.concatenate([p_ref[...], ph_ref[...]], axis=0)
    n_ext = jnp.concatenate([n_ref[...], nh_ref[...]], axis=0)
    _map_body(a_ext, p_ext, po_ref, row0, h, w)
    _map_body(a_ext, n_ext, no_ref, row0, h, w)


def kernel(anchor, positive, negative):
    h, w = anchor.shape
    br = min(_BR, h)
    nb = h // br
    halo_blocks = h // _HALO

    def main_spec():
        return pl.BlockSpec((br, w), lambda b: (b, 0))

    def halo_spec():
        return pl.BlockSpec(
            (_HALO, w),
            lambda b: (jnp.minimum((b + 1) * (br // _HALO), halo_blocks - 1), 0),
        )

    import functools
    body = functools.partial(_dc_kernel, h=h, w=w, br=br)
    pos, neg = pl.pallas_call(
        body,
        grid=(nb,),
        in_specs=[main_spec(), main_spec(), main_spec(),
                  halo_spec(), halo_spec(), halo_spec()],
        out_specs=[pl.BlockSpec((br, w), lambda b: (b, 0))] * 2,
        out_shape=[jax.ShapeDtypeStruct((h, w), jnp.float32)] * 2,
        compiler_params=pltpu.CompilerParams(
            dimension_semantics=("arbitrary",),
        ),
    )(anchor, positive, negative, anchor, positive, negative)
    return (pos, neg)


# speedup vs baseline: 9935.1694x; 1.8793x over previous
import functools

import jax
import jax.numpy as jnp
from jax.experimental import pallas as pl
from jax.experimental.pallas import tpu as pltpu

_BR = 128
_HALO = 8


def _roll(x, shift, axis):
    return pltpu.roll(x, shift % x.shape[axis], axis)


def _combine(bd, bv, cd, cv, use_max):
    better = (cd > bd) if use_max else (cd < bd)
    return jnp.where(better, cd, bd), jnp.where(better, cv, bv)


def _map_body(a_ext, c_ext, o_ref, row0, h, w):
    br = o_ref.shape[0]
    d_ext = jnp.abs(a_ext - c_ext)

    row_rel = jax.lax.broadcasted_iota(jnp.int32, (br, w), 0)
    col = jax.lax.broadcasted_iota(jnp.int32, (br, w), 1)
    even_r = (row_rel % 2) == 0
    even_c = (col % 2) == 0

    d0, d1, d2 = (d_ext[k:k + br] for k in range(3))
    c0, c1, c2 = (c_ext[k:k + br] for k in range(3))
    md, mv = _combine(d0, c0, d1, c1, False)
    md, mv = _combine(md, mv, d2, c2, False)
    xd, xv = _combine(d0, c0, d1, c1, True)
    xd, xv = _combine(xd, xv, d2, c2, True)

    md1, mv1 = _roll(md, -1, 1), _roll(mv, -1, 1)
    md2, mv2 = _roll(md, -2, 1), _roll(mv, -2, 1)
    xd1, xv1 = _roll(xd, -1, 1), _roll(xv, -1, 1)
    xd2, xv2 = _roll(xd, -2, 1), _roll(xv, -2, 1)
    md, mv = _combine(md, mv, md1, mv1, False)
    md, mv = _combine(md, mv, md2, mv2, False)
    xd, xv = _combine(xd, xv, xd1, xv1, True)
    xd, xv = _combine(xd, xv, xd2, xv2, True)

    out = mv + xv

    out = jnp.where(even_c, out, _roll(out, 1, 1))
    out = jnp.where(even_r, out, _roll(out, 1, 0))

    row_g = row_rel + row0
    out = jnp.where(col == w - 2, _roll(out, 2, 1), out)
    out = jnp.where(row_g == h - 2, _roll(out, 2, 0), out)
    out = jnp.where((row_g == h - 1) | (col == w - 1), 2.0 * c_ext[0:br], out)
    o_ref[...] = out


def _dc_kernel(a_ref, p_ref, n_ref, ah_ref, ph_ref, nh_ref, po_ref, no_ref,
               *, h, w, br):
    b = pl.program_id(0)
    row0 = b * br
    a_ext = jnp.concatenate([a_ref[...], ah_ref[...]], axis=0)
    p_ext = jnp.concatenate([p_ref[...], ph_ref[...]], axis=0)
    n_ext = jnp.concatenate([n_ref[...], nh_ref[...]], axis=0)
    _map_body(a_ext, p_ext, po_ref, row0, h, w)
    _map_body(a_ext, n_ext, no_ref, row0, h, w)


def kernel(anchor, positive, negative):
    h, w = anchor.shape
    br = min(_BR, h)
    nb = h // br
    halo_blocks = h // _HALO

    def main_spec():
        return pl.BlockSpec((br, w), lambda b: (b, 0))

    def halo_spec():
        return pl.BlockSpec(
            (_HALO, w),
            lambda b: (jnp.minimum((b + 1) * (br // _HALO), halo_blocks - 1), 0),
        )

    body = functools.partial(_dc_kernel, h=h, w=w, br=br)
    pos, neg = pl.pallas_call(
        body,
        grid=(nb,),
        in_specs=[main_spec(), main_spec(), main_spec(),
                  halo_spec(), halo_spec(), halo_spec()],
        out_specs=[pl.BlockSpec((br, w), lambda b: (b, 0))] * 2,
        out_shape=[jax.ShapeDtypeStruct((h, w), jnp.float32)] * 2,
        compiler_params=pltpu.CompilerParams(
            dimension_semantics=("arbitrary",),
        ),
    )(anchor, positive, negative, anchor, positive, negative)
    return (pos, neg)
